# baseline (speedup 1.0000x reference)
import functools

import jax
import jax.numpy as jnp
from jax import lax
from jax.experimental import pallas as pl
from jax.experimental.pallas import tpu as pltpu
from jax.experimental.pallas import tpu_sc as plsc

B = 4096
N = 8192
NC = 2
NS = 16
L = 16
NW = NC * NS
RPW = B // NW
K = 4
NCHUNK = RPW // K
LPR = N // 128
NBLK = RPW // 8

_mesh = plsc.VectorSubcoreMesh(core_axis_name="c", subcore_axis_name="s")


@functools.partial(
    pl.kernel,
    out_type=jax.ShapeDtypeStruct((B // 8, B // 128, 8, 128), jnp.float32),
    mesh=_mesh,
    compiler_params=pltpu.CompilerParams(needs_layout_passes=False,
                                         use_tc_tiling_on_sc=False),
    scratch_types=[
        pltpu.VMEM((B,), jnp.int32),
        pltpu.VMEM((NCHUNK, K * LPR), jnp.int32),
        pltpu.VMEM((2 * K * LPR, 128), jnp.float32),
        pltpu.VMEM((2, B // 128, K, 128), jnp.float32),
        pltpu.SemaphoreType.DMA,
        pltpu.SemaphoreType.DMA,
        pltpu.SemaphoreType.DMA,
        pltpu.SemaphoreType.DMA,
    ],
)
def _pairwise_sc(ri_hbm, gidx_hbm, compat_hbm, out_hbm,
                 ri_v, gidx_v, rowbuf, outbuf,
                 gsem0, gsem1, osem0, osem1):
    wid = lax.axis_index("s") * NC + lax.axis_index("c")
    gsems = (gsem0, gsem1)
    osems = (osem0, osem1)

    pltpu.sync_copy(ri_hbm, ri_v)
    pltpu.sync_copy(gidx_hbm.at[pl.ds(wid * NCHUNK, NCHUNK)], gidx_v)

    def start_gather(c, q):
        return pltpu.async_copy(compat_hbm.at[gidx_v.at[c]],
                                rowbuf.at[pl.ds(q * K * LPR, K * LPR)],
                                gsems[q])

    def compute_chunk(q, p):
        @plsc.parallel_loop(0, B // 256, unroll=2)
        def body(t):
            for u in range(16):
                riv = ri_v[pl.ds(t * 256 + u * L, L)]
                hi = lax.shift_right_logical(riv, 7)
                lo = lax.bitwise_and(riv, 127)
                for r in range(K):
                    idx0 = hi + (q * K * LPR + r * LPR)
                    vals = plsc.load_gather(rowbuf, [idx0, lo])
                    outbuf[p, t * 2 + u // 8, r, pl.ds((u % 8) * L, L)] = vals

    def out_dst(blk, q):
        return out_hbm.at[wid * NBLK + blk, :, pl.ds(q * K, K)]

    start_gather(0, 0)

    def outer(cc, carry):
        for q in range(2):
            c = 2 * cc + q

            @pl.when(c + 1 < NCHUNK)
            def _():
                start_gather(c + 1, 1 - q)

            pltpu.make_async_copy(compat_hbm.at[gidx_v.at[c]],
                                  rowbuf.at[pl.ds(q * K * LPR, K * LPR)],
                                  gsems[q]).wait()

            @pl.when(c >= 2)
            def _():
                pltpu.make_async_copy(outbuf.at[q], out_dst(cc - 1, q),
                                      osems[q]).wait()

            compute_chunk(q, q)
            pltpu.async_copy(outbuf.at[q], out_dst(cc, q), osems[q])
        return carry

    lax.fori_loop(0, NCHUNK // 2, outer, 0)
    for q in range(2):
        pltpu.make_async_copy(outbuf.at[q], out_dst(NCHUNK // 2 - 1, q),
                              osems[q]).wait()


def kernel(root_indices, compat_matrix):
    ri = root_indices.astype(jnp.int32)
    compat_lines = (compat_matrix.reshape(N // 8, 8, N // 128, 128)
                    .transpose(0, 2, 1, 3).reshape(N * N // 128, 128))
    t8 = jnp.arange(LPR, dtype=jnp.int32) * 8
    gidx = ((ri >> 3) * 512 + (ri & 7))[:, None] + t8[None, :]
    gidx = gidx.reshape(B // K, K * LPR)
    out4 = _pairwise_sc(ri, gidx, compat_lines)
    return out4.transpose(0, 2, 1, 3).reshape(B, B)


# speedup vs baseline: 1.1153x; 1.1153x over previous
import functools

import jax
import jax.numpy as jnp
from jax import lax
from jax.experimental import pallas as pl
from jax.experimental.pallas import tpu as pltpu
from jax.experimental.pallas import tpu_sc as plsc

B = 4096
N = 8192
NC = 2
NS = 16
L = 16
NW = NC * NS
RPW = B // NW
K = 4
NCHUNK = RPW // K
LPR = N // 128
NBLK = RPW // 8

_mesh = plsc.VectorSubcoreMesh(core_axis_name="c", subcore_axis_name="s")


@functools.partial(
    pl.kernel,
    out_type=jax.ShapeDtypeStruct((B // 8, B // 128, 8, 128), jnp.float32),
    mesh=_mesh,
    compiler_params=pltpu.CompilerParams(needs_layout_passes=False,
                                         use_tc_tiling_on_sc=False),
    scratch_types=[
        pltpu.VMEM((B,), jnp.int32),
        pltpu.VMEM((NCHUNK, K * LPR), jnp.int32),
        pltpu.VMEM((2 * K * LPR, 128), jnp.float32),
        pltpu.VMEM((2, B // 128, K, 128), jnp.float32),
        pltpu.SemaphoreType.DMA,
        pltpu.SemaphoreType.DMA,
        pltpu.SemaphoreType.DMA,
        pltpu.SemaphoreType.DMA,
    ],
)
def _pairwise_sc(ri_hbm, gidx_hbm, compat_hbm, out_hbm,
                 ri_v, gidx_v, rowbuf, outbuf,
                 gsem0, gsem1, osem0, osem1):
    wid = lax.axis_index("s") * NC + lax.axis_index("c")
    gsems = (gsem0, gsem1)
    osems = (osem0, osem1)

    pltpu.sync_copy(ri_hbm, ri_v)
    pltpu.sync_copy(gidx_hbm.at[pl.ds(wid * NCHUNK, NCHUNK)], gidx_v)

    def start_gather(c, q):
        return pltpu.async_copy(compat_hbm.at[gidx_v.at[c]],
                                rowbuf.at[pl.ds(q * K * LPR, K * LPR)],
                                gsems[q])

    def compute_chunk(q, p):
        @plsc.parallel_loop(0, B // 128, unroll=2)
        def body(t):
            for u in range(8):
                riv = ri_v[pl.ds(t * 128 + u * L, L)]
                hi = lax.shift_right_logical(riv, 7)
                lo = lax.bitwise_and(riv, 127)
                for r in range(K):
                    idx0 = hi + (q * K * LPR + r * LPR)
                    vals = plsc.load_gather(rowbuf, [idx0, lo])
                    outbuf[p, t, r, pl.ds(u * L, L)] = vals

    def out_dst(blk, q):
        return out_hbm.at[wid * NBLK + blk, :, pl.ds(q * K, K)]

    start_gather(0, 0)

    def outer(cc, carry):
        for q in range(2):
            c = 2 * cc + q

            @pl.when(c + 1 < NCHUNK)
            def _():
                start_gather(c + 1, 1 - q)

            pltpu.make_async_copy(compat_hbm.at[gidx_v.at[c]],
                                  rowbuf.at[pl.ds(q * K * LPR, K * LPR)],
                                  gsems[q]).wait()

            @pl.when(c >= 2)
            def _():
                pltpu.make_async_copy(outbuf.at[q], out_dst(cc - 1, q),
                                      osems[q]).wait()

            compute_chunk(q, q)
            pltpu.async_copy(outbuf.at[q], out_dst(cc, q), osems[q])
        return carry

    lax.fori_loop(0, NCHUNK // 2, outer, 0)
    for q in range(2):
        pltpu.make_async_copy(outbuf.at[q], out_dst(NCHUNK // 2 - 1, q),
                              osems[q]).wait()


def kernel(root_indices, compat_matrix):
    ri = root_indices.astype(jnp.int32)
    compat_lines = (compat_matrix.reshape(N // 8, 8, N // 128, 128)
                    .transpose(0, 2, 1, 3).reshape(N * N // 128, 128))
    t8 = jnp.arange(LPR, dtype=jnp.int32) * 8
    gidx = ((ri >> 3) * 512 + (ri & 7))[:, None] + t8[None, :]
    gidx = gidx.reshape(B // K, K * LPR)
    out4 = _pairwise_sc(ri, gidx, compat_lines)
    return out4.transpose(0, 2, 1, 3).reshape(B, B)
